# Initial kernel scaffold; baseline (speedup 1.0000x reference)
import jax
import jax.numpy as jnp
from jax.experimental import pallas as pl


def kernel(x, edge_index, W1_init, W1_root, b1, W2_init, W2_root, b2):
    raise NotImplementedError("write your pallas kernel here")


# speedup vs baseline: 19.6131x; 19.6131x over previous
import functools

import jax
import jax.numpy as jnp
from jax import lax
from jax.experimental import pallas as pl
from jax.experimental.pallas import tpu as pltpu, tpu_sc as plsc

NC = 2
NS = 16
NW = NC * NS

N_NODES = 10000
N_PAD = 10240
SENT = N_NODES
B = 128
RPT = N_PAD // NS


def _seg_kernel(D, with_gather):
    mesh = plsc.VectorSubcoreMesh(core_axis_name="c", subcore_axis_name="s")

    def body(*refs):
        if with_gather:
            (row_hbm, col_hbm, tab_hbm, zeros_hbm, out_hbm,
             rowv, colv, buf, agg_sh, sem) = refs
        else:
            (col_hbm, ones_hbm, zeros_hbm, out_hbm,
             colv, buf, agg_sh, sem) = refs
        c = lax.axis_index("c")
        s = lax.axis_index("s")
        wid = c * NS + s
        r0 = s * RPT
        pltpu.sync_copy(zeros_hbm.at[pl.ds(r0, RPT)], agg_sh.at[pl.ds(r0, RPT)])
        pltpu.sync_copy(col_hbm.at[wid], colv)
        if with_gather:
            pltpu.sync_copy(row_hbm.at[wid], rowv)
        else:
            pltpu.sync_copy(ones_hbm, buf)
        plsc.subcore_barrier()
        ch = colv.shape[0]

        def chunk(j, carry):
            if with_gather:
                pltpu.async_copy(tab_hbm.at[rowv.at[j]], buf, sem).wait()
            pltpu.sync_copy(buf, agg_sh.at[colv.at[j]], add=True)
            return carry

        lax.fori_loop(0, ch, chunk, 0)
        plsc.subcore_barrier()
        pltpu.sync_copy(agg_sh.at[pl.ds(r0, RPT)],
                        out_hbm.at[c].at[pl.ds(r0, RPT)])

    def make(ch):
        sc = []
        if with_gather:
            sc.append(pltpu.VMEM((ch, B), jnp.int32))
        sc.append(pltpu.VMEM((ch, B), jnp.int32))
        sc.append(pltpu.VMEM((B, D), jnp.float32))
        sc.append(pltpu.VMEM_SHARED((N_PAD, D), jnp.float32))
        sc.append(pltpu.SemaphoreType.DMA)
        return pl.kernel(
            body,
            out_type=jax.ShapeDtypeStruct((NC, N_PAD, D), jnp.float32),
            mesh=mesh,
            scratch_types=sc,
            compiler_params=pltpu.CompilerParams(use_tc_tiling_on_sc=False),
        )
    return make


def _tc_a(deg, x_pad, w1i, w1r, b1):
    R = 512
    grid = (N_PAD // R,)

    def body(deg_ref, x_ref, wi_ref, wr_ref, b_ref, hs_ref, root_ref, dis_ref):
        d = deg_ref[0] + deg_ref[1]
        dis = jnp.where(d > 0.0, lax.rsqrt(d), 0.0)
        xb = x_ref[...]
        h = jnp.dot(xb, wi_ref[...], preferred_element_type=jnp.float32)
        hs_ref[...] = h * dis
        root_ref[...] = (
            jnp.dot(xb, wr_ref[...], preferred_element_type=jnp.float32)
            + b_ref[...]
        )
        dis_ref[...] = dis

    H = w1i.shape[1]
    return pl.pallas_call(
        body,
        grid=grid,
        in_specs=[
            pl.BlockSpec((NC, R, H), lambda i: (0, i, 0)),
            pl.BlockSpec((R, x_pad.shape[1]), lambda i: (i, 0)),
            pl.BlockSpec(w1i.shape, lambda i: (0, 0)),
            pl.BlockSpec(w1r.shape, lambda i: (0, 0)),
            pl.BlockSpec((1, H), lambda i: (0, 0)),
        ],
        out_specs=[
            pl.BlockSpec((R, H), lambda i: (i, 0)),
            pl.BlockSpec((R, H), lambda i: (i, 0)),
            pl.BlockSpec((R, H), lambda i: (i, 0)),
        ],
        out_shape=[
            jax.ShapeDtypeStruct((N_PAD, H), jnp.float32),
            jax.ShapeDtypeStruct((N_PAD, H), jnp.float32),
            jax.ShapeDtypeStruct((N_PAD, H), jnp.float32),
        ],
    )(deg, x_pad, w1i, w1r, b1)


def _tc_b(agg1, dis16, root1, w2i, w2r, b2):
    R = 512
    grid = (N_PAD // R,)
    H = dis16.shape[1]
    D2 = w2i.shape[1]
    rep = D2 // H

    def body(agg_ref, dis_ref, root_ref, wi_ref, wr_ref, b_ref,
             hs_ref, root2_ref):
        dis = dis_ref[...]
        out1 = jnp.maximum(dis * (agg_ref[0] + agg_ref[1]) + root_ref[...], 0.0)
        disw = jnp.concatenate([dis] * rep, axis=1)
        h = jnp.dot(out1, wi_ref[...], preferred_element_type=jnp.float32)
        hs_ref[...] = h * disw
        root2_ref[...] = (
            jnp.dot(out1, wr_ref[...], preferred_element_type=jnp.float32)
            + b_ref[...]
        )

    return pl.pallas_call(
        body,
        grid=grid,
        in_specs=[
            pl.BlockSpec((NC, R, H), lambda i: (0, i, 0)),
            pl.BlockSpec((R, H), lambda i: (i, 0)),
            pl.BlockSpec((R, H), lambda i: (i, 0)),
            pl.BlockSpec(w2i.shape, lambda i: (0, 0)),
            pl.BlockSpec(w2r.shape, lambda i: (0, 0)),
            pl.BlockSpec((1, D2), lambda i: (0, 0)),
        ],
        out_specs=[
            pl.BlockSpec((R, D2), lambda i: (i, 0)),
            pl.BlockSpec((R, D2), lambda i: (i, 0)),
        ],
        out_shape=[
            jax.ShapeDtypeStruct((N_PAD, D2), jnp.float32),
            jax.ShapeDtypeStruct((N_PAD, D2), jnp.float32),
        ],
    )(agg1, dis16, root1, w2i, w2r, b2)


def _tc_c(agg2, dis16, root2):
    R = 512
    grid = (N_PAD // R,)
    H = dis16.shape[1]
    D2 = root2.shape[1]
    rep = D2 // H

    def body(agg_ref, dis_ref, root_ref, out_ref):
        disw = jnp.concatenate([dis_ref[...]] * rep, axis=1)
        out_ref[...] = jnp.maximum(
            disw * (agg_ref[0] + agg_ref[1]) + root_ref[...], 0.0)

    return pl.pallas_call(
        body,
        grid=grid,
        in_specs=[
            pl.BlockSpec((NC, R, D2), lambda i: (0, i, 0)),
            pl.BlockSpec((R, H), lambda i: (i, 0)),
            pl.BlockSpec((R, D2), lambda i: (i, 0)),
        ],
        out_specs=pl.BlockSpec((R, D2), lambda i: (i, 0)),
        out_shape=jax.ShapeDtypeStruct((N_PAD, D2), jnp.float32),
    )(agg2, dis16, root2)


@jax.jit
def _run(x, edge_index, W1_init, W1_root, b1, W2_init, W2_root, b2):
    N, F = x.shape
    E = edge_index.shape[1]
    H = W1_init.shape[1]
    C = W2_init.shape[1]
    D2 = 48

    ch = -(-E // (NW * B))
    e_pad = NW * ch * B

    row = jnp.concatenate(
        [edge_index[0], jnp.full((e_pad - E,), SENT, jnp.int32)]
    ).reshape(NW, ch, B)
    col = jnp.concatenate(
        [edge_index[1], jnp.full((e_pad - E,), SENT, jnp.int32)]
    ).reshape(NW, ch, B)

    x_pad = jnp.zeros((N_PAD, F), jnp.float32).at[:N].set(x)
    w2i = jnp.zeros((H, D2), jnp.float32).at[:, :C].set(W2_init)
    w2r = jnp.zeros((H, D2), jnp.float32).at[:, :C].set(W2_root)
    b2p = jnp.zeros((1, D2), jnp.float32).at[0, :C].set(b2)
    b1p = b1.reshape(1, H)

    zeros16 = jnp.zeros((N_PAD, H), jnp.float32)
    zeros48 = jnp.zeros((N_PAD, D2), jnp.float32)
    ones_b = jnp.ones((B, H), jnp.float32)

    deg = _seg_kernel(H, with_gather=False)(ch)(col, ones_b, zeros16)
    hs1, root1, dis16 = _tc_a(deg, x_pad, W1_init, W1_root, b1p)
    agg1 = _seg_kernel(H, with_gather=True)(ch)(row, col, hs1, zeros16)
    hs2, root2 = _tc_b(agg1, dis16, root1, w2i, w2r, b2p)
    agg2 = _seg_kernel(D2, with_gather=True)(ch)(row, col, hs2, zeros48)
    out = _tc_c(agg2, dis16, root2)
    return out[:N, :C]


def kernel(x, edge_index, W1_init, W1_root, b1, W2_init, W2_root, b2):
    return _run(x, edge_index, W1_init, W1_root, b1, W2_init, W2_root, b2)
